# baseline (speedup 1.0000x reference)
import functools

import jax
import jax.numpy as jnp
from jax import lax
from jax.experimental import pallas as pl
from jax.experimental.pallas import tpu as pltpu
from jax.experimental.pallas import tpu_sc as plsc

_VOCAB = 30522
_HIDDEN = 768
_MAX_POS = 512
_TYPE_VOCAB = 2
_BATCH = 10

_NTOK = _BATCH * _MAX_POS
_NC = 2
_NS = 16
_NW = _NC * _NS
_CHUNK = 32
_CHUNKS = _NTOK // _CHUNK
_CPW = _CHUNKS // _NW
_NSL = _HIDDEN // 16
_LANES = 16


def _emb_body(idx_hbm, seg_hbm, word_hbm, pos_hbm, type_hbm, gam_hbm, bet_hbm,
              out_hbm, idx_v, seg_v, wrows, prows, trows, gam_v, bet_v, sem):
    wid = lax.axis_index("s") * _NC + lax.axis_index("c")

    pltpu.sync_copy(type_hbm, trows)
    pltpu.sync_copy(gam_hbm, gam_v)
    pltpu.sync_copy(bet_hbm, bet_v)

    def chunk_body(j, carry):
        c = wid * _CPW + j
        base = c * _CHUNK
        p0 = lax.rem(base, _MAX_POS)

        pltpu.sync_copy(idx_hbm.at[pl.ds(base, _CHUNK)], idx_v)
        pltpu.sync_copy(seg_hbm.at[pl.ds(base, _CHUNK)],
                        seg_v.at[pl.ds(0, _CHUNK)])
        pltpu.async_copy(word_hbm.at[idx_v], wrows, sem).wait()
        pltpu.sync_copy(pos_hbm.at[pl.ds(p0, _CHUNK)], prows)

        def tok_body(t, carry2):
            sf = seg_v[pl.ds(t, _LANES)][0].astype(jnp.float32)

            def acc_body(k, acc_carry):
                acc, acc2 = acc_carry
                sl = pl.ds(k * _LANES, _LANES)
                w = wrows[t, sl]
                p = prows[t, sl]
                t0 = trows[0, sl]
                t1 = trows[1, sl]
                e = w + p + t0 + sf * (t1 - t0)
                wrows[t, sl] = e
                return acc + e, acc2 + e * e

            zero = jnp.zeros((_LANES,), jnp.float32)
            acc, acc2 = lax.fori_loop(0, _NSL, acc_body, (zero, zero))
            mean = jnp.sum(acc) * (1.0 / _HIDDEN)
            ex2 = jnp.sum(acc2) * (1.0 / _HIDDEN)
            var_eps = ex2 - mean * mean + 1e-5

            xv = jnp.full((_LANES,), var_eps, jnp.float32)
            yi = jnp.full((_LANES,), 0x5F3759DF, jnp.int32) - (
                plsc.bitcast(xv, jnp.int32) >> 1)
            y = plsc.bitcast(yi, jnp.float32)
            half_x = 0.5 * xv
            for _ in range(4):
                y = y * (1.5 - half_x * y * y)
            mean_v = jnp.full((_LANES,), mean, jnp.float32)

            def norm_body(k, _):
                sl = pl.ds(k * _LANES, _LANES)
                e = wrows[t, sl]
                wrows[t, sl] = (e - mean_v) * y * gam_v[sl] + bet_v[sl]
                return 0

            lax.fori_loop(0, _NSL, norm_body, 0)
            return carry2

        lax.fori_loop(0, _CHUNK, tok_body, 0)
        pltpu.sync_copy(wrows, out_hbm.at[pl.ds(base, _CHUNK)])
        return carry

    lax.fori_loop(0, _CPW, chunk_body, 0)


@jax.jit
def _emb_call(idx_flat, seg_flat, word_table, pos_table, type_table,
              ln_gamma, ln_beta):
    mesh = plsc.VectorSubcoreMesh(core_axis_name="c", subcore_axis_name="s")
    return pl.kernel(
        _emb_body,
        out_type=jax.ShapeDtypeStruct((_NTOK, _HIDDEN), jnp.float32),
        mesh=mesh,
        compiler_params=pltpu.CompilerParams(needs_layout_passes=False),
        scratch_types=[
            pltpu.VMEM((_CHUNK,), jnp.int32),
            pltpu.VMEM((_CHUNK + _LANES,), jnp.int32),
            pltpu.VMEM((_CHUNK, _HIDDEN), jnp.float32),
            pltpu.VMEM((_CHUNK, _HIDDEN), jnp.float32),
            pltpu.VMEM((_TYPE_VOCAB, _HIDDEN), jnp.float32),
            pltpu.VMEM((_HIDDEN,), jnp.float32),
            pltpu.VMEM((_HIDDEN,), jnp.float32),
            pltpu.SemaphoreType.DMA,
        ],
    )(idx_flat, seg_flat, word_table, pos_table, type_table, ln_gamma,
      ln_beta)


def kernel(batch_idx, batch_seg_idx, word_table, pos_table, type_table,
           ln_gamma, ln_beta):
    idx_flat = batch_idx.reshape(-1).astype(jnp.int32)
    seg_flat = batch_seg_idx.reshape(-1).astype(jnp.int32)
    out = _emb_call(idx_flat, seg_flat, word_table, pos_table, type_table,
                    ln_gamma, ln_beta)
    return out.reshape(_BATCH, _MAX_POS, _HIDDEN)


# speedup vs baseline: 1.1528x; 1.1528x over previous
import functools

import jax
import jax.numpy as jnp
from jax import lax
from jax.experimental import pallas as pl
from jax.experimental.pallas import tpu as pltpu
from jax.experimental.pallas import tpu_sc as plsc

_VOCAB = 30522
_HIDDEN = 768
_MAX_POS = 512
_TYPE_VOCAB = 2
_BATCH = 10

_NTOK = _BATCH * _MAX_POS
_NC = 2
_NS = 16
_NW = _NC * _NS
_CHUNK = 32
_CHUNKS = _NTOK // _CHUNK
_CPW = _CHUNKS // _NW
_NSL = _HIDDEN // 16
_LANES = 16


def _emb_body(idx_hbm, seg_hbm, word_hbm, pos_hbm, type_hbm, gam_hbm, bet_hbm,
              out_hbm, idx_v, seg_v, wrows, prows, trows, gam_v, bet_v, sem):
    wid = lax.axis_index("s") * _NC + lax.axis_index("c")

    pltpu.sync_copy(type_hbm, trows)
    pltpu.sync_copy(gam_hbm, gam_v)
    pltpu.sync_copy(bet_hbm, bet_v)

    def chunk_body(j, carry):
        c = wid * _CPW + j
        base = c * _CHUNK
        p0 = lax.rem(base, _MAX_POS)

        pltpu.sync_copy(idx_hbm.at[pl.ds(base, _CHUNK)], idx_v)
        pltpu.sync_copy(seg_hbm.at[pl.ds(base, _CHUNK)],
                        seg_v.at[pl.ds(0, _CHUNK)])
        pltpu.async_copy(word_hbm.at[idx_v], wrows, sem).wait()
        pltpu.sync_copy(pos_hbm.at[pl.ds(p0, _CHUNK)], prows)

        def tok_body(t, carry2):
            sf = seg_v[pl.ds(t, _LANES)][0].astype(jnp.float32)

            zero = jnp.zeros((_LANES,), jnp.float32)
            accs = [zero, zero, zero, zero]
            accs2 = [zero, zero, zero, zero]
            for k in range(_NSL):
                sl = pl.ds(k * _LANES, _LANES)
                w = wrows[t, sl]
                p = prows[t, sl]
                t0 = trows[0, sl]
                t1 = trows[1, sl]
                e = w + p + t0 + sf * (t1 - t0)
                wrows[t, sl] = e
                r = k % 4
                accs[r] = accs[r] + e
                accs2[r] = accs2[r] + e * e
            acc = (accs[0] + accs[1]) + (accs[2] + accs[3])
            acc2 = (accs2[0] + accs2[1]) + (accs2[2] + accs2[3])
            mean = jnp.sum(acc) * (1.0 / _HIDDEN)
            ex2 = jnp.sum(acc2) * (1.0 / _HIDDEN)
            var_eps = ex2 - mean * mean + 1e-5

            xv = jnp.full((_LANES,), var_eps, jnp.float32)
            yi = jnp.full((_LANES,), 0x5F3759DF, jnp.int32) - (
                plsc.bitcast(xv, jnp.int32) >> 1)
            y = plsc.bitcast(yi, jnp.float32)
            half_x = 0.5 * xv
            for _ in range(4):
                y = y * (1.5 - half_x * y * y)
            mean_v = jnp.full((_LANES,), mean, jnp.float32)

            for k in range(_NSL):
                sl = pl.ds(k * _LANES, _LANES)
                e = wrows[t, sl]
                wrows[t, sl] = (e - mean_v) * y * gam_v[sl] + bet_v[sl]
            return carry2

        lax.fori_loop(0, _CHUNK, tok_body, 0)
        pltpu.sync_copy(wrows, out_hbm.at[pl.ds(base, _CHUNK)])
        return carry

    lax.fori_loop(0, _CPW, chunk_body, 0)


@jax.jit
def _emb_call(idx_flat, seg_flat, word_table, pos_table, type_table,
              ln_gamma, ln_beta):
    mesh = plsc.VectorSubcoreMesh(core_axis_name="c", subcore_axis_name="s")
    return pl.kernel(
        _emb_body,
        out_type=jax.ShapeDtypeStruct((_NTOK, _HIDDEN), jnp.float32),
        mesh=mesh,
        compiler_params=pltpu.CompilerParams(needs_layout_passes=False),
        scratch_types=[
            pltpu.VMEM((_CHUNK,), jnp.int32),
            pltpu.VMEM((_CHUNK + _LANES,), jnp.int32),
            pltpu.VMEM((_CHUNK, _HIDDEN), jnp.float32),
            pltpu.VMEM((_CHUNK, _HIDDEN), jnp.float32),
            pltpu.VMEM((_TYPE_VOCAB, _HIDDEN), jnp.float32),
            pltpu.VMEM((_HIDDEN,), jnp.float32),
            pltpu.VMEM((_HIDDEN,), jnp.float32),
            pltpu.SemaphoreType.DMA,
        ],
    )(idx_flat, seg_flat, word_table, pos_table, type_table, ln_gamma,
      ln_beta)


def kernel(batch_idx, batch_seg_idx, word_table, pos_table, type_table,
           ln_gamma, ln_beta):
    idx_flat = batch_idx.reshape(-1).astype(jnp.int32)
    seg_flat = batch_seg_idx.reshape(-1).astype(jnp.int32)
    out = _emb_call(idx_flat, seg_flat, word_table, pos_table, type_table,
                    ln_gamma, ln_beta)
    return out.reshape(_BATCH, _MAX_POS, _HIDDEN)


# speedup vs baseline: 1.3778x; 1.1951x over previous
import functools

import jax
import jax.numpy as jnp
from jax import lax
from jax.experimental import pallas as pl
from jax.experimental.pallas import tpu as pltpu
from jax.experimental.pallas import tpu_sc as plsc

_VOCAB = 30522
_HIDDEN = 768
_MAX_POS = 512
_TYPE_VOCAB = 2
_BATCH = 10

_NTOK = _BATCH * _MAX_POS
_NC = 2
_NS = 16
_NW = _NC * _NS
_CHUNK = 32
_CHUNKS = _NTOK // _CHUNK
_CPW = _CHUNKS // _NW
_NSL = _HIDDEN // 16
_LANES = 16


def _emb_body(idx_hbm, seg_hbm, word_hbm, pos_hbm, type_hbm, gam_hbm, bet_hbm,
              out_hbm, idx_v, seg_v, wrows, prows, trows, gam_v, bet_v, sem):
    wid = lax.axis_index("s") * _NC + lax.axis_index("c")

    pltpu.sync_copy(type_hbm, trows)
    pltpu.sync_copy(gam_hbm, gam_v)
    pltpu.sync_copy(bet_hbm, bet_v)

    def chunk_body(j, carry):
        c = wid * _CPW + j
        base = c * _CHUNK
        p0 = lax.rem(base, _MAX_POS)

        pltpu.sync_copy(idx_hbm.at[pl.ds(base, _CHUNK)], idx_v)
        pltpu.sync_copy(seg_hbm.at[pl.ds(base, _CHUNK)],
                        seg_v.at[pl.ds(0, _CHUNK)])
        pltpu.async_copy(word_hbm.at[idx_v], wrows, sem).wait()
        pltpu.sync_copy(pos_hbm.at[pl.ds(p0, _CHUNK)], prows)

        def tok_body(u, carry2):
            ta = 2 * u
            tb = ta + 1
            sv = seg_v[pl.ds(ta, _LANES)]
            sfa = sv[0].astype(jnp.float32)
            sfb = sv[1].astype(jnp.float32)

            zero = jnp.zeros((_LANES,), jnp.float32)
            acca = [zero, zero]
            acca2 = [zero, zero]
            accb = [zero, zero]
            accb2 = [zero, zero]
            for k in range(_NSL):
                sl = pl.ds(k * _LANES, _LANES)
                t0 = trows[0, sl]
                t1 = trows[1, sl]
                d = t1 - t0
                wa = wrows[ta, sl]
                pa = prows[ta, sl]
                ea = wa + pa + t0 + sfa * d
                wb = wrows[tb, sl]
                pb = prows[tb, sl]
                eb = wb + pb + t0 + sfb * d
                wrows[ta, sl] = ea
                wrows[tb, sl] = eb
                r = k % 2
                acca[r] = acca[r] + ea
                acca2[r] = acca2[r] + ea * ea
                accb[r] = accb[r] + eb
                accb2[r] = accb2[r] + eb * eb
            mean_a = jnp.sum(acca[0] + acca[1]) * (1.0 / _HIDDEN)
            ex2_a = jnp.sum(acca2[0] + acca2[1]) * (1.0 / _HIDDEN)
            mean_b = jnp.sum(accb[0] + accb[1]) * (1.0 / _HIDDEN)
            ex2_b = jnp.sum(accb2[0] + accb2[1]) * (1.0 / _HIDDEN)
            va = ex2_a - mean_a * mean_a + 1e-5
            vb = ex2_b - mean_b * mean_b + 1e-5

            xa = jnp.full((_LANES,), va, jnp.float32)
            xb = jnp.full((_LANES,), vb, jnp.float32)
            magic = jnp.full((_LANES,), 0x5F3759DF, jnp.int32)
            ya = plsc.bitcast(magic - (plsc.bitcast(xa, jnp.int32) >> 1),
                              jnp.float32)
            yb = plsc.bitcast(magic - (plsc.bitcast(xb, jnp.int32) >> 1),
                              jnp.float32)
            hxa = 0.5 * xa
            hxb = 0.5 * xb
            for _ in range(4):
                ya = ya * (1.5 - hxa * ya * ya)
                yb = yb * (1.5 - hxb * yb * yb)
            ma = jnp.full((_LANES,), mean_a, jnp.float32)
            mb = jnp.full((_LANES,), mean_b, jnp.float32)

            for k in range(_NSL):
                sl = pl.ds(k * _LANES, _LANES)
                g = gam_v[sl]
                b = bet_v[sl]
                ea = wrows[ta, sl]
                eb = wrows[tb, sl]
                wrows[ta, sl] = (ea - ma) * ya * g + b
                wrows[tb, sl] = (eb - mb) * yb * g + b
            return carry2

        lax.fori_loop(0, _CHUNK // 2, tok_body, 0)
        pltpu.sync_copy(wrows, out_hbm.at[pl.ds(base, _CHUNK)])
        return carry

    lax.fori_loop(0, _CPW, chunk_body, 0)


@jax.jit
def _emb_call(idx_flat, seg_flat, word_table, pos_table, type_table,
              ln_gamma, ln_beta):
    mesh = plsc.VectorSubcoreMesh(core_axis_name="c", subcore_axis_name="s")
    return pl.kernel(
        _emb_body,
        out_type=jax.ShapeDtypeStruct((_NTOK, _HIDDEN), jnp.float32),
        mesh=mesh,
        compiler_params=pltpu.CompilerParams(needs_layout_passes=False),
        scratch_types=[
            pltpu.VMEM((_CHUNK,), jnp.int32),
            pltpu.VMEM((_CHUNK + _LANES,), jnp.int32),
            pltpu.VMEM((_CHUNK, _HIDDEN), jnp.float32),
            pltpu.VMEM((_CHUNK, _HIDDEN), jnp.float32),
            pltpu.VMEM((_TYPE_VOCAB, _HIDDEN), jnp.float32),
            pltpu.VMEM((_HIDDEN,), jnp.float32),
            pltpu.VMEM((_HIDDEN,), jnp.float32),
            pltpu.SemaphoreType.DMA,
        ],
    )(idx_flat, seg_flat, word_table, pos_table, type_table, ln_gamma,
      ln_beta)


def kernel(batch_idx, batch_seg_idx, word_table, pos_table, type_table,
           ln_gamma, ln_beta):
    idx_flat = batch_idx.reshape(-1).astype(jnp.int32)
    seg_flat = batch_seg_idx.reshape(-1).astype(jnp.int32)
    out = _emb_call(idx_flat, seg_flat, word_table, pos_table, type_table,
                    ln_gamma, ln_beta)
    return out.reshape(_BATCH, _MAX_POS, _HIDDEN)


# speedup vs baseline: 1.5601x; 1.1323x over previous
import functools

import jax
import jax.numpy as jnp
from jax import lax
from jax.experimental import pallas as pl
from jax.experimental.pallas import tpu as pltpu
from jax.experimental.pallas import tpu_sc as plsc

_VOCAB = 30522
_HIDDEN = 768
_MAX_POS = 512
_TYPE_VOCAB = 2
_BATCH = 10

_NTOK = _BATCH * _MAX_POS
_NC = 2
_NS = 16
_NW = _NC * _NS
_CHUNK = 32
_CHUNKS = _NTOK // _CHUNK
_CPW = _CHUNKS // _NW
_NSL = _HIDDEN // 16
_LANES = 16


def _emb_body(idx_hbm, seg_hbm, word_hbm, pos_hbm, type_hbm, gam_hbm, bet_hbm,
              out_hbm, idx_v, seg_v, wrows, prows, trows, gam_v, bet_v, sem):
    wid = lax.axis_index("s") * _NC + lax.axis_index("c")

    pltpu.sync_copy(type_hbm, trows)
    pltpu.sync_copy(gam_hbm, gam_v)
    pltpu.sync_copy(bet_hbm, bet_v)

    def chunk_body(j, carry):
        c = wid * _CPW + j
        base = c * _CHUNK
        p0 = lax.rem(base, _MAX_POS)

        pltpu.sync_copy(idx_hbm.at[pl.ds(base, _CHUNK)], idx_v)
        pltpu.sync_copy(seg_hbm.at[pl.ds(base, _CHUNK)],
                        seg_v.at[pl.ds(0, _CHUNK)])
        pltpu.async_copy(word_hbm.at[idx_v], wrows, sem).wait()
        pltpu.sync_copy(pos_hbm.at[pl.ds(p0, _CHUNK)], prows)

        def tok_body(u, carry2):
            ta = 2 * u
            tb = ta + 1
            sv = seg_v[pl.ds(ta, _LANES)]
            sfa = sv[0].astype(jnp.float32)
            sfb = sv[1].astype(jnp.float32)

            zero = jnp.zeros((_LANES,), jnp.float32)

            def p1_block(kb, carry):
                acca = [carry[0], carry[1]]
                acca2 = [carry[2], carry[3]]
                accb = [carry[4], carry[5]]
                accb2 = [carry[6], carry[7]]
                base_k = kb * (12 * _LANES)
                for kk in range(12):
                    sl = pl.ds(base_k + kk * _LANES, _LANES)
                    t0 = trows[0, sl]
                    t1 = trows[1, sl]
                    d = t1 - t0
                    wa = wrows[ta, sl]
                    pa = prows[ta, sl]
                    ea = wa + pa + t0 + sfa * d
                    wb = wrows[tb, sl]
                    pb = prows[tb, sl]
                    eb = wb + pb + t0 + sfb * d
                    wrows[ta, sl] = ea
                    wrows[tb, sl] = eb
                    r = kk % 2
                    acca[r] = acca[r] + ea
                    acca2[r] = acca2[r] + ea * ea
                    accb[r] = accb[r] + eb
                    accb2[r] = accb2[r] + eb * eb
                return (acca[0], acca[1], acca2[0], acca2[1],
                        accb[0], accb[1], accb2[0], accb2[1])

            cf = lax.fori_loop(0, 4, p1_block, (zero,) * 8)
            mean_a = jnp.sum(cf[0] + cf[1]) * (1.0 / _HIDDEN)
            ex2_a = jnp.sum(cf[2] + cf[3]) * (1.0 / _HIDDEN)
            mean_b = jnp.sum(cf[4] + cf[5]) * (1.0 / _HIDDEN)
            ex2_b = jnp.sum(cf[6] + cf[7]) * (1.0 / _HIDDEN)
            va = ex2_a - mean_a * mean_a + 1e-5
            vb = ex2_b - mean_b * mean_b + 1e-5

            xa = jnp.full((_LANES,), va, jnp.float32)
            xb = jnp.full((_LANES,), vb, jnp.float32)
            magic = jnp.full((_LANES,), 0x5F3759DF, jnp.int32)
            ya = plsc.bitcast(magic - (plsc.bitcast(xa, jnp.int32) >> 1),
                              jnp.float32)
            yb = plsc.bitcast(magic - (plsc.bitcast(xb, jnp.int32) >> 1),
                              jnp.float32)
            hxa = 0.5 * xa
            hxb = 0.5 * xb
            for _ in range(4):
                ya = ya * (1.5 - hxa * ya * ya)
                yb = yb * (1.5 - hxb * yb * yb)
            ma = jnp.full((_LANES,), mean_a, jnp.float32)
            mb = jnp.full((_LANES,), mean_b, jnp.float32)

            for k in range(_NSL):
                sl = pl.ds(k * _LANES, _LANES)
                g = gam_v[sl]
                b = bet_v[sl]
                ea = wrows[ta, sl]
                eb = wrows[tb, sl]
                wrows[ta, sl] = (ea - ma) * ya * g + b
                wrows[tb, sl] = (eb - mb) * yb * g + b
            return carry2

        lax.fori_loop(0, _CHUNK // 2, tok_body, 0)
        pltpu.sync_copy(wrows, out_hbm.at[pl.ds(base, _CHUNK)])
        return carry

    lax.fori_loop(0, _CPW, chunk_body, 0)


@jax.jit
def _emb_call(idx_flat, seg_flat, word_table, pos_table, type_table,
              ln_gamma, ln_beta):
    mesh = plsc.VectorSubcoreMesh(core_axis_name="c", subcore_axis_name="s")
    return pl.kernel(
        _emb_body,
        out_type=jax.ShapeDtypeStruct((_NTOK, _HIDDEN), jnp.float32),
        mesh=mesh,
        compiler_params=pltpu.CompilerParams(needs_layout_passes=False),
        scratch_types=[
            pltpu.VMEM((_CHUNK,), jnp.int32),
            pltpu.VMEM((_CHUNK + _LANES,), jnp.int32),
            pltpu.VMEM((_CHUNK, _HIDDEN), jnp.float32),
            pltpu.VMEM((_CHUNK, _HIDDEN), jnp.float32),
            pltpu.VMEM((_TYPE_VOCAB, _HIDDEN), jnp.float32),
            pltpu.VMEM((_HIDDEN,), jnp.float32),
            pltpu.VMEM((_HIDDEN,), jnp.float32),
            pltpu.SemaphoreType.DMA,
        ],
    )(idx_flat, seg_flat, word_table, pos_table, type_table, ln_gamma,
      ln_beta)


def kernel(batch_idx, batch_seg_idx, word_table, pos_table, type_table,
           ln_gamma, ln_beta):
    idx_flat = batch_idx.reshape(-1).astype(jnp.int32)
    seg_flat = batch_seg_idx.reshape(-1).astype(jnp.int32)
    out = _emb_call(idx_flat, seg_flat, word_table, pos_table, type_table,
                    ln_gamma, ln_beta)
    return out.reshape(_BATCH, _MAX_POS, _HIDDEN)


# speedup vs baseline: 1.6758x; 1.0741x over previous
import functools

import jax
import jax.numpy as jnp
from jax import lax
from jax.experimental import pallas as pl
from jax.experimental.pallas import tpu as pltpu
from jax.experimental.pallas import tpu_sc as plsc

_VOCAB = 30522
_HIDDEN = 768
_MAX_POS = 512
_TYPE_VOCAB = 2
_BATCH = 10

_NC = 2
_NS = 16
_NW = _NC * _NS
_CHUNK = 32
_CPW = 5
_NSL = _HIDDEN // 16
_LANES = 16


def _emb_body(idx_hbm, seg_hbm, word_hbm, pos_hbm, type_hbm, gam_hbm, bet_hbm,
              out_hbm, idx_v, seg_v, wrows0, wrows1, prows, trows, gam_v,
              bet_v, isem, gsem0, gsem1, osem0, osem1):
    wid = lax.axis_index("s") * _NC + lax.axis_index("c")
    p0 = (wid % _NS) * _CHUNK
    b0 = (wid // _NS) * _CPW

    id_descs = []
    for j in range(_CPW):
        base_j = (b0 + j) * _MAX_POS + p0
        id_descs.append(pltpu.async_copy(
            idx_hbm.at[pl.ds(base_j, _CHUNK)], idx_v.at[j], isem))
        id_descs.append(pltpu.async_copy(
            seg_hbm.at[pl.ds(base_j, _CHUNK)],
            seg_v.at[j, pl.ds(0, _CHUNK)], isem))
    pltpu.sync_copy(pos_hbm.at[pl.ds(p0, _CHUNK)], prows)
    pltpu.sync_copy(type_hbm, trows)
    pltpu.sync_copy(gam_hbm, gam_v)
    pltpu.sync_copy(bet_hbm, bet_v)
    for d in id_descs:
        d.wait()

    bufs = ((wrows0, gsem0, osem0), (wrows1, gsem1, osem1))

    def compute_chunk(j, wrows):
        def tok_body(u, carry2):
            ta = 2 * u
            tb = ta + 1
            sv = seg_v[j, pl.ds(ta, _LANES)]
            sfa = sv[0].astype(jnp.float32)
            sfb = sv[1].astype(jnp.float32)

            zero = jnp.zeros((_LANES,), jnp.float32)

            def p1_block(kb, carry):
                acca = [carry[0], carry[1]]
                acca2 = [carry[2], carry[3]]
                accb = [carry[4], carry[5]]
                accb2 = [carry[6], carry[7]]
                base_k = kb * (12 * _LANES)
                for kk in range(12):
                    sl = pl.ds(base_k + kk * _LANES, _LANES)
                    t0 = trows[0, sl]
                    t1 = trows[1, sl]
                    d = t1 - t0
                    wa = wrows[ta, sl]
                    pa = prows[ta, sl]
                    ea = wa + pa + t0 + sfa * d
                    wb = wrows[tb, sl]
                    pb = prows[tb, sl]
                    eb = wb + pb + t0 + sfb * d
                    wrows[ta, sl] = ea
                    wrows[tb, sl] = eb
                    r = kk % 2
                    acca[r] = acca[r] + ea
                    acca2[r] = acca2[r] + ea * ea
                    accb[r] = accb[r] + eb
                    accb2[r] = accb2[r] + eb * eb
                return (acca[0], acca[1], acca2[0], acca2[1],
                        accb[0], accb[1], accb2[0], accb2[1])

            cf = lax.fori_loop(0, 4, p1_block, (zero,) * 8)
            mean_a = jnp.sum(cf[0] + cf[1]) * (1.0 / _HIDDEN)
            ex2_a = jnp.sum(cf[2] + cf[3]) * (1.0 / _HIDDEN)
            mean_b = jnp.sum(cf[4] + cf[5]) * (1.0 / _HIDDEN)
            ex2_b = jnp.sum(cf[6] + cf[7]) * (1.0 / _HIDDEN)
            va = ex2_a - mean_a * mean_a + 1e-5
            vb = ex2_b - mean_b * mean_b + 1e-5

            xa = jnp.full((_LANES,), va, jnp.float32)
            xb = jnp.full((_LANES,), vb, jnp.float32)
            magic = jnp.full((_LANES,), 0x5F3759DF, jnp.int32)
            ya = plsc.bitcast(magic - (plsc.bitcast(xa, jnp.int32) >> 1),
                              jnp.float32)
            yb = plsc.bitcast(magic - (plsc.bitcast(xb, jnp.int32) >> 1),
                              jnp.float32)
            hxa = 0.5 * xa
            hxb = 0.5 * xb
            for _ in range(4):
                ya = ya * (1.5 - hxa * ya * ya)
                yb = yb * (1.5 - hxb * yb * yb)
            ma = jnp.full((_LANES,), mean_a, jnp.float32)
            mb = jnp.full((_LANES,), mean_b, jnp.float32)

            for k in range(_NSL):
                sl = pl.ds(k * _LANES, _LANES)
                g = gam_v[sl]
                b = bet_v[sl]
                ea = wrows[ta, sl]
                eb = wrows[tb, sl]
                wrows[ta, sl] = (ea - ma) * ya * g + b
                wrows[tb, sl] = (eb - mb) * yb * g + b
            return carry2

        lax.fori_loop(0, _CHUNK // 2, tok_body, 0)

    gather_desc = [None] * _CPW
    out_desc = [None] * _CPW
    gather_desc[0] = pltpu.async_copy(word_hbm.at[idx_v.at[0]], wrows0,
                                      gsem0)
    for j in range(_CPW):
        wr, _, os_ = bufs[j % 2]
        if j + 1 < _CPW:
            nwr, ngs, _ = bufs[(j + 1) % 2]
            if j >= 1:
                out_desc[j - 1].wait()
            gather_desc[j + 1] = pltpu.async_copy(
                word_hbm.at[idx_v.at[j + 1]], nwr, ngs)
        gather_desc[j].wait()
        compute_chunk(j, wr)
        out_base = (b0 + j) * _MAX_POS + p0
        out_desc[j] = pltpu.async_copy(
            wr, out_hbm.at[pl.ds(out_base, _CHUNK)], os_)
    out_desc[_CPW - 2].wait()
    out_desc[_CPW - 1].wait()


@jax.jit
def _emb_call(batch_idx, batch_seg_idx, word_table, pos_table, type_table,
              ln_gamma, ln_beta):
    mesh = plsc.VectorSubcoreMesh(core_axis_name="c", subcore_axis_name="s")
    return pl.kernel(
        _emb_body,
        out_type=jax.ShapeDtypeStruct((_BATCH * _MAX_POS, _HIDDEN),
                                      jnp.float32),
        mesh=mesh,
        compiler_params=pltpu.CompilerParams(needs_layout_passes=False),
        scratch_types=[
            pltpu.VMEM((_CPW, _CHUNK), jnp.int32),
            pltpu.VMEM((_CPW, 2 * _CHUNK), jnp.int32),
            pltpu.VMEM((_CHUNK, _HIDDEN), jnp.float32),
            pltpu.VMEM((_CHUNK, _HIDDEN), jnp.float32),
            pltpu.VMEM((_CHUNK, _HIDDEN), jnp.float32),
            pltpu.VMEM((_TYPE_VOCAB, _HIDDEN), jnp.float32),
            pltpu.VMEM((_HIDDEN,), jnp.float32),
            pltpu.VMEM((_HIDDEN,), jnp.float32),
            pltpu.SemaphoreType.DMA,
            pltpu.SemaphoreType.DMA,
            pltpu.SemaphoreType.DMA,
            pltpu.SemaphoreType.DMA,
            pltpu.SemaphoreType.DMA,
        ],
    )(batch_idx, batch_seg_idx, word_table, pos_table, type_table, ln_gamma,
      ln_beta)


def kernel(batch_idx, batch_seg_idx, word_table, pos_table, type_table,
           ln_gamma, ln_beta):
    idx_flat = batch_idx.reshape(-1).astype(jnp.int32)
    seg_flat = batch_seg_idx.reshape(-1).astype(jnp.int32)
    out = _emb_call(idx_flat, seg_flat, word_table, pos_table, type_table,
                    ln_gamma, ln_beta)
    return out.reshape(_BATCH, _MAX_POS, _HIDDEN)


# speedup vs baseline: 3.0182x; 1.8010x over previous
import functools

import jax
import jax.numpy as jnp
from jax import lax
from jax.experimental import pallas as pl
from jax.experimental.pallas import tpu as pltpu
from jax.experimental.pallas import tpu_sc as plsc

_VOCAB = 30522
_HIDDEN = 768
_MAX_POS = 512
_TYPE_VOCAB = 2
_BATCH = 10

_NC = 2
_NS = 16
_NW = _NC * _NS
_CHUNK = 32
_CPW = 5
_NSL = _HIDDEN // 16
_LANES = 16


def _emb_body(idx_hbm, seg_hbm, word_hbm, pos_hbm, type_hbm, gam_hbm, bet_hbm,
              out_hbm, idx_v, seg_v, wrows0, wrows1, prows, trows, gam_v,
              bet_v, isem, gsem0, gsem1, osem0, osem1):
    wid = lax.axis_index("s") * _NC + lax.axis_index("c")
    p0 = (wid % _NS) * _CHUNK
    b0 = (wid // _NS) * _CPW

    id_descs = []
    for j in range(_CPW):
        base_j = (b0 + j) * _MAX_POS + p0
        id_descs.append(pltpu.async_copy(
            idx_hbm.at[pl.ds(base_j, _CHUNK)], idx_v.at[j], isem))
        id_descs.append(pltpu.async_copy(
            seg_hbm.at[pl.ds(base_j, _CHUNK)],
            seg_v.at[j, pl.ds(0, _CHUNK)], isem))
    pltpu.sync_copy(pos_hbm.at[pl.ds(p0, _CHUNK)], prows)
    pltpu.sync_copy(type_hbm, trows)
    pltpu.sync_copy(gam_hbm, gam_v)
    pltpu.sync_copy(bet_hbm, bet_v)
    for d in id_descs:
        d.wait()

    bufs = ((wrows0, gsem0, osem0), (wrows1, gsem1, osem1))

    def compute_chunk(j, wrows):
        def tok_body(u, carry2):
            ta = 2 * u
            tb = ta + 1
            sv = seg_v[j, pl.ds(ta, _LANES)]
            sfa = sv[0].astype(jnp.float32)
            sfb = sv[1].astype(jnp.float32)

            zero = jnp.zeros((_LANES,), jnp.float32)

            @plsc.parallel_loop(0, _HIDDEN, _LANES, unroll=4,
                                carry=(zero,) * 4)
            def p1_cf(i, carry):
                sl = pl.ds(i, _LANES)
                t0 = trows[0, sl]
                t1 = trows[1, sl]
                d = t1 - t0
                wa = wrows[ta, sl]
                pa = prows[ta, sl]
                ea = wa + pa + t0 + sfa * d
                wb = wrows[tb, sl]
                pb = prows[tb, sl]
                eb = wb + pb + t0 + sfb * d
                wrows[ta, sl] = ea
                wrows[tb, sl] = eb
                return (carry[0] + ea, carry[1] + ea * ea,
                        carry[2] + eb, carry[3] + eb * eb)

            mean_a = jnp.sum(p1_cf[0]) * (1.0 / _HIDDEN)
            ex2_a = jnp.sum(p1_cf[1]) * (1.0 / _HIDDEN)
            mean_b = jnp.sum(p1_cf[2]) * (1.0 / _HIDDEN)
            ex2_b = jnp.sum(p1_cf[3]) * (1.0 / _HIDDEN)
            va = ex2_a - mean_a * mean_a + 1e-5
            vb = ex2_b - mean_b * mean_b + 1e-5

            xa = jnp.full((_LANES,), va, jnp.float32)
            xb = jnp.full((_LANES,), vb, jnp.float32)
            magic = jnp.full((_LANES,), 0x5F3759DF, jnp.int32)
            ya = plsc.bitcast(magic - (plsc.bitcast(xa, jnp.int32) >> 1),
                              jnp.float32)
            yb = plsc.bitcast(magic - (plsc.bitcast(xb, jnp.int32) >> 1),
                              jnp.float32)
            hxa = 0.5 * xa
            hxb = 0.5 * xb
            for _ in range(4):
                ya = ya * (1.5 - hxa * ya * ya)
                yb = yb * (1.5 - hxb * yb * yb)
            ma = jnp.full((_LANES,), mean_a, jnp.float32)
            mb = jnp.full((_LANES,), mean_b, jnp.float32)

            @plsc.parallel_loop(0, _HIDDEN, _LANES, unroll=4)
            def _p2(i):
                sl = pl.ds(i, _LANES)
                g = gam_v[sl]
                b = bet_v[sl]
                ea = wrows[ta, sl]
                eb = wrows[tb, sl]
                wrows[ta, sl] = (ea - ma) * ya * g + b
                wrows[tb, sl] = (eb - mb) * yb * g + b
            return carry2

        lax.fori_loop(0, _CHUNK // 2, tok_body, 0)

    gather_desc = [None] * _CPW
    out_desc = [None] * _CPW
    gather_desc[0] = pltpu.async_copy(word_hbm.at[idx_v.at[0]], wrows0,
                                      gsem0)
    for j in range(_CPW):
        wr, _, os_ = bufs[j % 2]
        if j + 1 < _CPW:
            nwr, ngs, _ = bufs[(j + 1) % 2]
            if j >= 1:
                out_desc[j - 1].wait()
            gather_desc[j + 1] = pltpu.async_copy(
                word_hbm.at[idx_v.at[j + 1]], nwr, ngs)
        gather_desc[j].wait()
        compute_chunk(j, wr)
        out_base = (b0 + j) * _MAX_POS + p0
        out_desc[j] = pltpu.async_copy(
            wr, out_hbm.at[pl.ds(out_base, _CHUNK)], os_)
    out_desc[_CPW - 2].wait()
    out_desc[_CPW - 1].wait()


@jax.jit
def _emb_call(batch_idx, batch_seg_idx, word_table, pos_table, type_table,
              ln_gamma, ln_beta):
    mesh = plsc.VectorSubcoreMesh(core_axis_name="c", subcore_axis_name="s")
    return pl.kernel(
        _emb_body,
        out_type=jax.ShapeDtypeStruct((_BATCH * _MAX_POS, _HIDDEN),
                                      jnp.float32),
        mesh=mesh,
        compiler_params=pltpu.CompilerParams(needs_layout_passes=False),
        scratch_types=[
            pltpu.VMEM((_CPW, _CHUNK), jnp.int32),
            pltpu.VMEM((_CPW, 2 * _CHUNK), jnp.int32),
            pltpu.VMEM((_CHUNK, _HIDDEN), jnp.float32),
            pltpu.VMEM((_CHUNK, _HIDDEN), jnp.float32),
            pltpu.VMEM((_CHUNK, _HIDDEN), jnp.float32),
            pltpu.VMEM((_TYPE_VOCAB, _HIDDEN), jnp.float32),
            pltpu.VMEM((_HIDDEN,), jnp.float32),
            pltpu.VMEM((_HIDDEN,), jnp.float32),
            pltpu.SemaphoreType.DMA,
            pltpu.SemaphoreType.DMA,
            pltpu.SemaphoreType.DMA,
            pltpu.SemaphoreType.DMA,
            pltpu.SemaphoreType.DMA,
        ],
    )(batch_idx, batch_seg_idx, word_table, pos_table, type_table, ln_gamma,
      ln_beta)


def kernel(batch_idx, batch_seg_idx, word_table, pos_table, type_table,
           ln_gamma, ln_beta):
    idx_flat = batch_idx.reshape(-1).astype(jnp.int32)
    seg_flat = batch_seg_idx.reshape(-1).astype(jnp.int32)
    out = _emb_call(idx_flat, seg_flat, word_table, pos_table, type_table,
                    ln_gamma, ln_beta)
    return out.reshape(_BATCH, _MAX_POS, _HIDDEN)
